# baseline (speedup 1.0000x reference)
import functools

import jax
import jax.numpy as jnp
from jax import lax
from jax.experimental import pallas as pl
from jax.experimental.pallas import tpu as pltpu
from jax.experimental.pallas import tpu_sc as plsc

B, H, W = 8, 512, 512
GH, GW, GD = 16, 16, 8
NC = 12
ZC = GD * NC
N_OUT = 3

ROWS_PW = (B * H) // 32
RH = 16
CW = 256
NG = CW // 16
XHALF = CW * ZC


def _fsplat(x):
    return jnp.full((16,), x, dtype=jnp.float32)


def _isplat(x):
    return jnp.full((16,), x, dtype=jnp.int32)


def _body(grid_hbm, guide_hbm, inp_hbm, out_hbm,
          grows_v, x_v, guide_v, inp_v, out_v):
    wid = lax.axis_index("s") * 2 + lax.axis_index("c")
    b = wid // 4
    row_base = (wid % 4) * ROWS_PW

    lane = lax.iota(jnp.int32, 16)
    lane3 = lane * 3
    laneZC = lane * ZC

    def outer(t, _):
        hb = t // 2
        ch = t % 2
        r0 = row_base + hb * RH
        m = r0 // RH
        jy0 = jnp.clip((m + 1) // 2 - 1, 0, GH - 1)
        jy1 = jnp.clip((m + 1) // 2, 0, GH - 1)
        c0 = ch * CW

        pltpu.sync_copy(grid_hbm.at[b, jy0], grows_v.at[pl.ds(0, GW * ZC)])
        pltpu.sync_copy(grid_hbm.at[b, jy1],
                        grows_v.at[pl.ds(GW * ZC, GW * ZC)])

        def stage_w(w, _):
            wg = c0 + w
            x0 = (wg + 16) // 32 - 1
            x0c = jnp.clip(x0, 0, GW - 1)
            x1c = jnp.clip(x0 + 1, 0, GW - 1)
            wx = (_fsplat(wg.astype(jnp.float32)) + 0.5) * (1.0 / 32.0) \
                - 0.5 - _fsplat(x0.astype(jnp.float32))
            for y01 in range(2):
                gb = y01 * (GW * ZC)
                for k in range(ZC // 16):
                    g0 = grows_v[pl.ds(gb + x0c * ZC + k * 16, 16)]
                    g1 = grows_v[pl.ds(gb + x1c * ZC + k * 16, 16)]
                    x_v[pl.ds(y01 * XHALF + w * ZC + k * 16, 16)] = \
                        g0 + wx * (g1 - g0)
            return 0

        lax.fori_loop(0, CW, stage_w, 0)

        def dma_in(r, _):
            pltpu.sync_copy(guide_hbm.at[b, r0 + r, pl.ds(c0, CW)],
                            guide_v.at[pl.ds(r * CW, CW)])
            pltpu.sync_copy(inp_hbm.at[b, r0 + r, pl.ds(c0 * 3, CW * 3)],
                            inp_v.at[pl.ds(r * CW * 3, CW * 3)])
            return 0

        lax.fori_loop(0, RH, dma_in, 0)

        def row_body(r, _):
            hrow = r0 + r
            gy = (_fsplat(hrow.astype(jnp.float32)) + 0.5) * (1.0 / 32.0) \
                - 0.5
            a1 = gy - ((gy + 1.0).astype(jnp.int32).astype(jnp.float32)
                       - 1.0)
            a0 = 1.0 - a1

            def grp_body(g, _):
                gv = guide_v[pl.ds(r * CW + g * 16, 16)]
                gz = jnp.clip(gv * GD - 0.5, 0.0, GD - 1.0)
                z0 = gz.astype(jnp.int32)
                f = gz - z0.astype(jnp.float32)
                z1 = jnp.minimum(z0 + 1, GD - 1)
                base = _isplat(g * 16 * ZC) + laneZC
                i0 = base + z0 * NC
                i1 = base + z1 * NC
                i0p = i0 + XHALF
                i1p = i1 + XHALF

                ib = _isplat(r * CW * 3 + g * 48) + lane3
                aug = [plsc.load_gather(inp_v, [ib + i]) for i in range(3)]

                for o in range(N_OUT):
                    cb = 4 * o
                    acc = None
                    for i in range(4):
                        v00 = plsc.load_gather(x_v, [i0 + (cb + i)])
                        v01 = plsc.load_gather(x_v, [i1 + (cb + i)])
                        v10 = plsc.load_gather(x_v, [i0p + (cb + i)])
                        v11 = plsc.load_gather(x_v, [i1p + (cb + i)])
                        cz = a0 * (v00 + f * (v01 - v00)) \
                            + a1 * (v10 + f * (v11 - v10))
                        if i == 3:
                            acc = acc + cz
                        else:
                            t2 = cz * aug[i]
                            acc = t2 if acc is None else acc + t2
                    plsc.store_scatter(out_v, [ib + o], acc)
                return 0

            lax.fori_loop(0, NG, grp_body, 0)
            return 0

        lax.fori_loop(0, RH, row_body, 0)

        def dma_out(r, _):
            pltpu.sync_copy(out_v.at[pl.ds(r * CW * 3, CW * 3)],
                            out_hbm.at[b, r0 + r, pl.ds(c0 * 3, CW * 3)])
            return 0

        lax.fori_loop(0, RH, dma_out, 0)
        return 0

    lax.fori_loop(0, (ROWS_PW // RH) * 2, outer, 0)


@jax.jit
def _run(grid, guide, inp):
    grid_r = grid.reshape(B, GH, GW * ZC)
    inp_f = inp.reshape(B, H, W * 3)
    mesh = plsc.VectorSubcoreMesh(core_axis_name="c", subcore_axis_name="s")
    kfn = functools.partial(
        pl.kernel,
        out_type=jax.ShapeDtypeStruct((B, H, W * 3), jnp.float32),
        mesh=mesh,
        compiler_params=pltpu.CompilerParams(
            needs_layout_passes=False, use_tc_tiling_on_sc=False),
        scratch_types=[
            pltpu.VMEM((2 * GW * ZC,), jnp.float32),
            pltpu.VMEM((2 * XHALF,), jnp.float32),
            pltpu.VMEM((RH * CW,), jnp.float32),
            pltpu.VMEM((RH * CW * 3,), jnp.float32),
            pltpu.VMEM((RH * CW * 3,), jnp.float32),
        ],
    )(_body)
    return kfn(grid_r, guide, inp_f).reshape(B, H, W, N_OUT)


def kernel(bilateral_grid, guide, input):
    return _run(bilateral_grid, guide, input)


# speedup vs baseline: 4.2928x; 4.2928x over previous
import functools

import jax
import jax.numpy as jnp
import numpy as np
from jax.experimental import pallas as pl

B, H, W = 8, 512, 512
GH, GW, GD = 16, 16, 8
N_IN = 3
N_COEF = 12
ZC = GD * N_COEF
TH = 16


def _ax_table() -> np.ndarray:
    w = np.arange(W, dtype=np.float64)
    gx = (w + 0.5) * GW / W - 0.5
    x0 = np.floor(gx).astype(np.int64)
    wx = (gx - x0).astype(np.float64)
    t = np.zeros((GW, W), dtype=np.float64)
    np.add.at(t, (np.clip(x0, 0, GW - 1), np.arange(W)), 1.0 - wx)
    np.add.at(t, (np.clip(x0 + 1, 0, GW - 1), np.arange(W)), wx)
    return t.astype(np.float32)


def _xinterp_body(g_ref, ax_ref, q_ref):
    q_ref[0] = jax.lax.dot_general(
        g_ref[0], ax_ref[...], (((1,), (0,)), ((), ())),
        preferred_element_type=jnp.float32)


def _slice_body(q0_ref, q1_ref, guide_ref, inp_ref, out_ref):
    m = pl.program_id(1)
    hi = jax.lax.broadcasted_iota(jnp.int32, (TH, W), 0)
    h = (m * TH + hi).astype(jnp.float32) + 0.5
    gy = h * (GH / H) - 0.5
    a1 = gy - jnp.floor(gy)

    q0 = q0_ref[0, 0].reshape(GD, N_COEF, 1, W)
    dq = q1_ref[0, 0].reshape(GD, N_COEF, 1, W) - q0
    pz = q0 + a1[None, None] * dq

    g = guide_ref[0]
    gz = jnp.clip(g * GD - 0.5, 0.0, GD - 1.0)
    coeff = None
    for z in range(GD):
        tz = jnp.maximum(0.0, 1.0 - jnp.abs(gz - float(z)))
        contrib = tz[None] * pz[z]
        coeff = contrib if coeff is None else coeff + contrib

    inp = inp_ref[0]
    for o in range(N_COEF // (N_IN + 1)):
        acc = coeff[4 * o + N_IN]
        for i in range(N_IN):
            acc = acc + coeff[4 * o + i] * inp[i]
        out_ref[0, o] = acc


@jax.jit
def _run(grid, guide, inp):
    n_out = N_COEF // (N_IN + 1)
    gt = jnp.transpose(grid, (0, 1, 3, 4, 2)).reshape(B, GH * ZC, GW)
    ax = jnp.asarray(_ax_table())

    q = pl.pallas_call(
        _xinterp_body,
        grid=(B,),
        in_specs=[
            pl.BlockSpec((1, GH * ZC, GW), lambda b: (b, 0, 0)),
            pl.BlockSpec((GW, W), lambda b: (0, 0)),
        ],
        out_specs=pl.BlockSpec((1, GH * ZC, W), lambda b: (b, 0, 0)),
        out_shape=jax.ShapeDtypeStruct((B, GH * ZC, W), jnp.float32),
    )(gt, ax)
    q = q.reshape(B, GH, ZC, W)

    inp_p = jnp.transpose(inp, (0, 3, 1, 2))

    def jy0(b, m):
        return jnp.clip((m + 1) // 2 - 1, 0, GH - 1)

    def jy1(b, m):
        return jnp.clip((m + 1) // 2, 0, GH - 1)

    out_p = pl.pallas_call(
        _slice_body,
        grid=(B, H // TH),
        in_specs=[
            pl.BlockSpec((1, 1, ZC, W), lambda b, m: (b, jy0(b, m), 0, 0)),
            pl.BlockSpec((1, 1, ZC, W), lambda b, m: (b, jy1(b, m), 0, 0)),
            pl.BlockSpec((1, TH, W), lambda b, m: (b, m, 0)),
            pl.BlockSpec((1, N_IN, TH, W), lambda b, m: (b, 0, m, 0)),
        ],
        out_specs=pl.BlockSpec((1, n_out, TH, W), lambda b, m: (b, 0, m, 0)),
        out_shape=jax.ShapeDtypeStruct((B, n_out, H, W), jnp.float32),
    )(q, q, guide, inp_p)

    return jnp.transpose(out_p, (0, 2, 3, 1))


def kernel(bilateral_grid, guide, input):
    return _run(bilateral_grid, guide, input)
